# baseline (speedup 1.0000x reference)
import functools

import jax
import jax.numpy as jnp
import numpy as np
from jax import lax
from jax.experimental import pallas as pl
from jax.experimental.pallas import tpu as pltpu
from jax.experimental.pallas import tpu_sc as plsc

D_MODEL = 256
D_FLOW = 128
D_FFN = 1024
NL = 3
NH = 8
NP = 4
_SHAPES = ((96, 96), (48, 48), (24, 24))
_STARTS = (0, 9216, 11520)
NB = 2
LQ = 9216
LIN = 12096
DH = D_MODEL // NH
GCOLS = NH * NL * NP
GPQ = 4 * GCOLS

_l_of_col = np.array([(c % 12) // 4 for c in range(GCOLS)])
_W_COL = np.array([_SHAPES[l][1] for l in _l_of_col], np.float32)[None, :]
_H_COL = np.array([_SHAPES[l][0] for l in _l_of_col], np.float32)[None, :]
_WI_COL = _W_COL.astype(np.int32)
_ST_COL = np.array([_STARTS[l] for l in _l_of_col], np.int32)[None, :]
_HCOL = np.array([c // 12 for c in range(GCOLS)], np.int32)[None, :]
_PERM = np.zeros((D_MODEL,), np.int64)
for _h in range(NH):
    for _c in range(16):
        _PERM[_h * DH + 2 * _c] = _h * DH + _c
        _PERM[_h * DH + 2 * _c + 1] = _h * DH + 16 + _c

_SMAT = np.zeros((GCOLS, GCOLS), np.float32)
for _i in range(GCOLS):
    for _j in range(GCOLS):
        if _i // 12 == _j // 12:
            _SMAT[_i, _j] = 1.0

_BV = 1008


def _value_body(src_ref, wv_ref, bv_ref, out_ref):
    out_ref[...] = (
        jnp.dot(src_ref[...], wv_ref[...], preferred_element_type=jnp.float32)
        + bv_ref[...]
    ).astype(jnp.bfloat16)


def _value_proj(src_flat, Wv, bv):
    return pl.pallas_call(
        _value_body,
        grid=(NB * LIN // _BV,),
        in_specs=[
            pl.BlockSpec((_BV, D_MODEL), lambda i: (i, 0)),
            pl.BlockSpec((D_MODEL, D_MODEL), lambda i: (0, 0)),
            pl.BlockSpec((1, D_MODEL), lambda i: (0, 0)),
        ],
        out_specs=pl.BlockSpec((_BV, D_MODEL), lambda i: (i, 0)),
        out_shape=jax.ShapeDtypeStruct((NB * LIN, D_MODEL), jnp.bfloat16),
    )(src_flat, Wv, bv.reshape(1, D_MODEL))


_BQ = 512


def _index_body(flow_ref, wox_ref, box_ref, woy_ref, boy_ref, wat_ref,
                bat_ref, smat_ref, wl_ref, hl_ref, wli_ref, st_ref, hc_ref,
                idx_ref, w_ref, *, qoff_b):
    b = pl.program_id(0)
    qb = pl.program_id(1) + qoff_b
    fl = flow_ref[0]
    offx = jnp.dot(fl, wox_ref[...], preferred_element_type=jnp.float32) + box_ref[...]
    offy = jnp.dot(fl, woy_ref[...], preferred_element_type=jnp.float32) + boy_ref[...]
    att = jnp.dot(fl, wat_ref[...], preferred_element_type=jnp.float32) + bat_ref[...]
    m = jnp.max(att, axis=1, keepdims=True)
    e = jnp.exp(att - m)
    s = jnp.dot(e, smat_ref[...], preferred_element_type=jnp.float32)
    aw = e / s

    q = qb * _BQ + lax.broadcasted_iota(jnp.int32, (_BQ, GCOLS), 0)
    xq = (q % 96).astype(jnp.float32)
    yq = (q // 96).astype(jnp.float32)
    wl = wl_ref[...]
    hl = hl_ref[...]
    px = (xq + 0.5) * (wl / 96.0) + offx - 0.5
    py = (yq + 0.5) * (hl / 96.0) + offy - 0.5
    fx = jnp.floor(px)
    tx = px - fx
    fy = jnp.floor(py)
    ty = py - fy
    wlm1 = wl - 1.0
    hlm1 = hl - 1.0
    vx0 = ((fx >= 0.0) & (fx <= wlm1)).astype(jnp.float32)
    vx1 = ((fx + 1.0 >= 0.0) & (fx + 1.0 <= wlm1)).astype(jnp.float32)
    vy0 = ((fy >= 0.0) & (fy <= hlm1)).astype(jnp.float32)
    vy1 = ((fy + 1.0 >= 0.0) & (fy + 1.0 <= hlm1)).astype(jnp.float32)
    ix0 = jnp.clip(fx, 0.0, wlm1).astype(jnp.int32)
    ix1 = jnp.clip(fx + 1.0, 0.0, wlm1).astype(jnp.int32)
    iy0 = jnp.clip(fy, 0.0, hlm1).astype(jnp.int32)
    iy1 = jnp.clip(fy + 1.0, 0.0, hlm1).astype(jnp.int32)
    wx0 = (1.0 - tx) * vx0
    wx1 = tx * vx1
    wy0 = (1.0 - ty) * vy0
    wy1 = ty * vy1

    wli = wli_ref[...]
    base = b * LIN + st_ref[...]
    hc = hc_ref[...]
    corners = ((ix0, iy0, wx0, wy0), (ix1, iy0, wx1, wy0),
               (ix0, iy1, wx0, wy1), (ix1, iy1, wx1, wy1))
    for c, (ix, iy, wx, wy) in enumerate(corners):
        idx_ref[0, :, c * GCOLS:(c + 1) * GCOLS] = (base + iy * wli + ix) * NH + hc
        wb = (aw * wx * wy).astype(jnp.bfloat16)
        wu = lax.bitcast_convert_type(wb, jnp.uint16).astype(jnp.uint32)
        w_ref[0, :, c * GCOLS:(c + 1) * GCOLS] = wu * jnp.uint32(65537)


def _index_weights(flow, W_off, b_off, W_att, b_att, qoff_b, nq):
    wox = W_off[:, 0::2]
    woy = W_off[:, 1::2]
    box = b_off[0::2].reshape(1, GCOLS)
    boy = b_off[1::2].reshape(1, GCOLS)
    consts = [jnp.asarray(_W_COL), jnp.asarray(_H_COL), jnp.asarray(_WI_COL),
              jnp.asarray(_ST_COL), jnp.asarray(_HCOL)]
    cspecs = [pl.BlockSpec((1, GCOLS), lambda b, i: (0, 0)) for _ in consts]
    return pl.pallas_call(
        functools.partial(_index_body, qoff_b=qoff_b),
        grid=(NB, nq // _BQ),
        in_specs=[
            pl.BlockSpec((1, _BQ, D_FLOW), lambda b, i: (b, i + qoff_b, 0)),
            pl.BlockSpec((D_FLOW, GCOLS), lambda b, i: (0, 0)),
            pl.BlockSpec((1, GCOLS), lambda b, i: (0, 0)),
            pl.BlockSpec((D_FLOW, GCOLS), lambda b, i: (0, 0)),
            pl.BlockSpec((1, GCOLS), lambda b, i: (0, 0)),
            pl.BlockSpec((D_FLOW, GCOLS), lambda b, i: (0, 0)),
            pl.BlockSpec((1, GCOLS), lambda b, i: (0, 0)),
            pl.BlockSpec((GCOLS, GCOLS), lambda b, i: (0, 0)),
        ] + cspecs,
        out_specs=[
            pl.BlockSpec((1, _BQ, GPQ), lambda b, i: (b, i, 0)),
            pl.BlockSpec((1, _BQ, GPQ), lambda b, i: (b, i, 0)),
        ],
        out_shape=[
            jax.ShapeDtypeStruct((NB, nq, GPQ), jnp.int32),
            jax.ShapeDtypeStruct((NB, nq, GPQ), jnp.uint32),
        ],
    )(flow, wox, box, woy, boy, W_att, b_att.reshape(1, GCOLS), jnp.asarray(_SMAT),
      *consts)


_NTILES = 32
_QC = 8
_CHUNKS = NB * LQ // _QC
_CPW = _CHUNKS // _NTILES
_NIDX = _QC * GPQ
_NSEG = _NIDX // 128


def _sc_body(value_hbm, idx_hbm, w_hbm, out_hbm, idx_v, w_v, rows_v, out_v,
             g0, g1, a0, a1, o0, o1, *, cpw):
    cid = lax.axis_index("c")
    sid = lax.axis_index("s")
    wid = sid * 2 + cid
    gsems = (g0, g1)
    asems = (a0, a1)
    osems = (o0, o1)

    def fetch_aw(slot, i, start):
        ci = wid * cpw + i
        idx_cp = (idx_hbm.at[ci], idx_v.at[pl.ds(slot * _NSEG, _NSEG)],
                  asems[slot])
        w_cp = (w_hbm.at[ci], w_v.at[pl.ds(slot * _NIDX, _NIDX)], asems[slot])
        if start:
            pltpu.async_copy(*idx_cp)
            pltpu.async_copy(*w_cp)
        else:
            pltpu.make_async_copy(*idx_cp).wait()
            pltpu.make_async_copy(*w_cp).wait()

    def gathers(slot, start):
        for j in range(_NSEG):
            cp = (value_hbm.at[idx_v.at[slot * _NSEG + j]],
                  rows_v.at[pl.ds((slot * _NIDX) + j * 128, 128)],
                  gsems[slot])
            if start:
                pltpu.async_copy(*cp)
            else:
                pltpu.make_async_copy(*cp).wait()

    def out_cp(slot, i, start):
        q0 = (wid * cpw + i) * _QC
        cp = (out_v.at[slot], out_hbm.at[pl.ds(q0, _QC)], osems[slot])
        if start:
            pltpu.async_copy(*cp)
        else:
            pltpu.make_async_copy(*cp).wait()

    def compute(slot, i):
        for qi in range(_QC):
            def h_body(h, hc):
                base = slot * _NIDX + qi * GPQ + h * 12
                acc0s, acc1s = [], []
                for c in range(4):
                    wv = w_v[pl.ds(base + c * GCOLS, 16)]
                    a0_ = [jnp.zeros((16,), jnp.float32) for _ in range(2)]
                    a1_ = [jnp.zeros((16,), jnp.float32) for _ in range(2)]
                    for pp in range(3):
                        p = 4 * pp
                        ws = [plsc.bitcast(jnp.take_along_axis(
                            wv, jnp.full((16,), p + j, jnp.int32), axis=0),
                            jnp.bfloat16) for j in range(4)]
                        r0 = base + c * GCOLS + p
                        s = ((rows_v[r0, :] * ws[0]
                              + rows_v[r0 + 1, :] * ws[1])
                             + (rows_v[r0 + 2, :] * ws[2]
                                + rows_v[r0 + 3, :] * ws[3]))
                        va, vb = plsc.unpack(
                            s, format=plsc.PackFormat.INTERLEAVED)
                        a0_[pp % 2] = a0_[pp % 2] + va
                        a1_[pp % 2] = a1_[pp % 2] + vb
                    acc0s.append(a0_[0] + a0_[1])
                    acc1s.append(a1_[0] + a1_[1])
                out_v[slot, qi, pl.ds(h * DH, 16)] = (
                    (acc0s[0] + acc0s[1]) + (acc0s[2] + acc0s[3]))
                out_v[slot, qi, pl.ds(h * DH + 16, 16)] = (
                    (acc1s[0] + acc1s[1]) + (acc1s[2] + acc1s[3]))
                return hc
            lax.fori_loop(0, NH, h_body, 0)

    fetch_aw(0, 0, True)
    fetch_aw(0, 0, False)
    gathers(0, True)
    fetch_aw(1, 1, True)

    def body(k, carry):
        i0 = k * 2
        for b in range(2):
            cur = i0 + b

            @pl.when(cur + 1 < cpw)
            def _():
                fetch_aw(1 - b, cur + 1, False)
                gathers(1 - b, True)

            gathers(b, False)

            @pl.when(cur >= 2)
            def _():
                out_cp(b, cur - 2, False)

            compute(b, cur)
            out_cp(b, cur, True)

            @pl.when(cur + 2 < cpw)
            def _():
                fetch_aw(b, cur + 2, True)
        return carry

    lax.fori_loop(0, cpw // 2, body, 0)
    out_cp(0, cpw - 2, False)
    out_cp(1, cpw - 1, False)


def _sc_sample(value_flat, idx_seg, w_flat, nq):
    cpw = nq // _QC // _NTILES
    mesh = plsc.VectorSubcoreMesh(core_axis_name="c", subcore_axis_name="s",
                                  num_cores=2, num_subcores=16)
    f = functools.partial(
        pl.kernel,
        out_type=jax.ShapeDtypeStruct((nq, D_MODEL), jnp.float32),
        mesh=mesh,
        compiler_params=pltpu.CompilerParams(needs_layout_passes=False,
                                             use_tc_tiling_on_sc=False),
        scratch_types=[
            pltpu.VMEM((2 * _NSEG, 128), jnp.int32),
            pltpu.VMEM((2 * _NIDX + 16,), jnp.uint32),
            pltpu.VMEM((2 * _NIDX, DH), jnp.bfloat16),
            pltpu.VMEM((2, _QC, D_MODEL), jnp.float32),
            pltpu.SemaphoreType.DMA,
            pltpu.SemaphoreType.DMA,
            pltpu.SemaphoreType.DMA,
            pltpu.SemaphoreType.DMA,
            pltpu.SemaphoreType.DMA,
            pltpu.SemaphoreType.DMA,
        ],
    )(functools.partial(_sc_body, cpw=cpw))
    return f(value_flat, idx_seg, w_flat)


_BO = 512


def _tail_body(x_ref, wo_ref, bo_ref, g_ref, be_ref, w1_ref, b1_ref, w2_ref,
               b2_ref, out_ref):
    src2 = jnp.dot(x_ref[...], wo_ref[...], preferred_element_type=jnp.float32) + bo_ref[...]
    mu = jnp.mean(src2, axis=1, keepdims=True)
    var = jnp.mean((src2 - mu) ** 2, axis=1, keepdims=True)
    ln = (src2 - mu) * lax.rsqrt(var + 1e-5) * g_ref[...] + be_ref[...]
    h1 = jnp.maximum(
        jnp.dot(ln, w1_ref[...], preferred_element_type=jnp.float32) + b1_ref[...], 0.0)
    ffn = jnp.dot(h1, w2_ref[...], preferred_element_type=jnp.float32) + b2_ref[...]
    out_ref[...] = src2 + ffn


def _tail(samp, Wo, bo, gamma, beta, W1, b1, W2, b2):
    return pl.pallas_call(
        _tail_body,
        grid=(samp.shape[0] // _BO,),
        in_specs=[
            pl.BlockSpec((_BO, D_MODEL), lambda i: (i, 0)),
            pl.BlockSpec((D_MODEL, D_MODEL), lambda i: (0, 0)),
            pl.BlockSpec((1, D_MODEL), lambda i: (0, 0)),
            pl.BlockSpec((1, D_MODEL), lambda i: (0, 0)),
            pl.BlockSpec((1, D_MODEL), lambda i: (0, 0)),
            pl.BlockSpec((D_MODEL, D_FFN), lambda i: (0, 0)),
            pl.BlockSpec((1, D_FFN), lambda i: (0, 0)),
            pl.BlockSpec((D_FFN, D_MODEL), lambda i: (0, 0)),
            pl.BlockSpec((1, D_MODEL), lambda i: (0, 0)),
        ],
        out_specs=pl.BlockSpec((_BO, D_MODEL), lambda i: (i, 0)),
        out_shape=jax.ShapeDtypeStruct((samp.shape[0], D_MODEL), jnp.float32),
    )(samp, Wo, bo.reshape(1, D_MODEL), gamma.reshape(1, D_MODEL),
      beta.reshape(1, D_MODEL), W1, b1.reshape(1, D_FFN), W2,
      b2.reshape(1, D_MODEL))


def kernel(src, flow, spatial_shapes, level_start_index, Wv, bv, W_off, b_off,
           W_att, b_att, Wo, bo, gamma, beta, W1, b1, W2, b2):
    value = _value_proj(src.reshape(NB * LIN, D_MODEL), Wv[:, _PERM],
                        bv[_PERM])
    value_flat = value.reshape(NB * LIN * NH, DH)
    hq = LQ // 2
    halves = []
    for qoff_b in (0, hq // _BQ):
        idx, w = _index_weights(flow, W_off, b_off, W_att, b_att, qoff_b, hq)
        chunks = NB * hq // _QC
        samp = _sc_sample(value_flat, idx.reshape(chunks, _NSEG, 128),
                          w.reshape(chunks, _NIDX), NB * hq)
        halves.append(_tail(samp, Wo, bo, gamma, beta, W1, b1, W2, b2)
                      .reshape(NB, hq, D_MODEL))
    return jnp.concatenate(halves, axis=1)


# speedup vs baseline: 1.0908x; 1.0908x over previous
import functools

import jax
import jax.numpy as jnp
import numpy as np
from jax import lax
from jax.experimental import pallas as pl
from jax.experimental.pallas import tpu as pltpu
from jax.experimental.pallas import tpu_sc as plsc

D_MODEL = 256
D_FLOW = 128
D_FFN = 1024
NL = 3
NH = 8
NP = 4
_SHAPES = ((96, 96), (48, 48), (24, 24))
_STARTS = (0, 9216, 11520)
NB = 2
LQ = 9216
LIN = 12096
DH = D_MODEL // NH
GCOLS = NH * NL * NP
GPQ = 4 * GCOLS

_l_of_col = np.array([(c % 12) // 4 for c in range(GCOLS)])
_W_COL = np.array([_SHAPES[l][1] for l in _l_of_col], np.float32)[None, :]
_H_COL = np.array([_SHAPES[l][0] for l in _l_of_col], np.float32)[None, :]
_WI_COL = _W_COL.astype(np.int32)
_ST_COL = np.array([_STARTS[l] for l in _l_of_col], np.int32)[None, :]
_HCOL = np.array([c // 12 for c in range(GCOLS)], np.int32)[None, :]
_PERM = np.zeros((D_MODEL,), np.int64)
for _h in range(NH):
    for _c in range(16):
        _PERM[_h * DH + 2 * _c] = _h * DH + _c
        _PERM[_h * DH + 2 * _c + 1] = _h * DH + 16 + _c

_SMAT = np.zeros((GCOLS, GCOLS), np.float32)
for _i in range(GCOLS):
    for _j in range(GCOLS):
        if _i // 12 == _j // 12:
            _SMAT[_i, _j] = 1.0

_BV = 1008


def _value_body(src_ref, wv_ref, bv_ref, out_ref):
    out_ref[...] = (
        jnp.dot(src_ref[...], wv_ref[...], preferred_element_type=jnp.float32)
        + bv_ref[...]
    ).astype(jnp.bfloat16)


def _value_proj(src_flat, Wv, bv):
    return pl.pallas_call(
        _value_body,
        grid=(NB * LIN // _BV,),
        in_specs=[
            pl.BlockSpec((_BV, D_MODEL), lambda i: (i, 0)),
            pl.BlockSpec((D_MODEL, D_MODEL), lambda i: (0, 0)),
            pl.BlockSpec((1, D_MODEL), lambda i: (0, 0)),
        ],
        out_specs=pl.BlockSpec((_BV, D_MODEL), lambda i: (i, 0)),
        out_shape=jax.ShapeDtypeStruct((NB * LIN, D_MODEL), jnp.bfloat16),
    )(src_flat, Wv, bv.reshape(1, D_MODEL))


_BQ = 512


def _index_body(flow_ref, wox_ref, box_ref, woy_ref, boy_ref, wat_ref,
                bat_ref, smat_ref, wl_ref, hl_ref, wli_ref, st_ref, hc_ref,
                idx_ref, w_ref, *, qoff_b):
    b = pl.program_id(0)
    qb = pl.program_id(1) + qoff_b
    fl = flow_ref[0]
    offx = jnp.dot(fl, wox_ref[...], preferred_element_type=jnp.float32) + box_ref[...]
    offy = jnp.dot(fl, woy_ref[...], preferred_element_type=jnp.float32) + boy_ref[...]
    att = jnp.dot(fl, wat_ref[...], preferred_element_type=jnp.float32) + bat_ref[...]
    m = jnp.max(att, axis=1, keepdims=True)
    e = jnp.exp(att - m)
    s = jnp.dot(e, smat_ref[...], preferred_element_type=jnp.float32)
    aw = e / s

    q = qb * _BQ + lax.broadcasted_iota(jnp.int32, (_BQ, GCOLS), 0)
    xq = (q % 96).astype(jnp.float32)
    yq = (q // 96).astype(jnp.float32)
    wl = wl_ref[...]
    hl = hl_ref[...]
    px = (xq + 0.5) * (wl / 96.0) + offx - 0.5
    py = (yq + 0.5) * (hl / 96.0) + offy - 0.5
    fx = jnp.floor(px)
    tx = px - fx
    fy = jnp.floor(py)
    ty = py - fy
    wlm1 = wl - 1.0
    hlm1 = hl - 1.0
    vx0 = ((fx >= 0.0) & (fx <= wlm1)).astype(jnp.float32)
    vx1 = ((fx + 1.0 >= 0.0) & (fx + 1.0 <= wlm1)).astype(jnp.float32)
    vy0 = ((fy >= 0.0) & (fy <= hlm1)).astype(jnp.float32)
    vy1 = ((fy + 1.0 >= 0.0) & (fy + 1.0 <= hlm1)).astype(jnp.float32)
    ix0 = jnp.clip(fx, 0.0, wlm1).astype(jnp.int32)
    ix1 = jnp.clip(fx + 1.0, 0.0, wlm1).astype(jnp.int32)
    iy0 = jnp.clip(fy, 0.0, hlm1).astype(jnp.int32)
    iy1 = jnp.clip(fy + 1.0, 0.0, hlm1).astype(jnp.int32)
    wx0 = (1.0 - tx) * vx0
    wx1 = tx * vx1
    wy0 = (1.0 - ty) * vy0
    wy1 = ty * vy1

    wli = wli_ref[...]
    base = b * LIN + st_ref[...]
    hc = hc_ref[...]
    corners = ((ix0, iy0, wx0, wy0), (ix1, iy0, wx1, wy0),
               (ix0, iy1, wx0, wy1), (ix1, iy1, wx1, wy1))
    for c, (ix, iy, wx, wy) in enumerate(corners):
        idx_ref[0, :, c * GCOLS:(c + 1) * GCOLS] = (base + iy * wli + ix) * NH + hc
        wb = (aw * wx * wy).astype(jnp.bfloat16)
        wu = lax.bitcast_convert_type(wb, jnp.uint16).astype(jnp.uint32)
        w_ref[0, :, c * GCOLS:(c + 1) * GCOLS] = wu * jnp.uint32(65537)


def _index_weights(flow, W_off, b_off, W_att, b_att, qoff_b, nq):
    wox = W_off[:, 0::2]
    woy = W_off[:, 1::2]
    box = b_off[0::2].reshape(1, GCOLS)
    boy = b_off[1::2].reshape(1, GCOLS)
    consts = [jnp.asarray(_W_COL), jnp.asarray(_H_COL), jnp.asarray(_WI_COL),
              jnp.asarray(_ST_COL), jnp.asarray(_HCOL)]
    cspecs = [pl.BlockSpec((1, GCOLS), lambda b, i: (0, 0)) for _ in consts]
    return pl.pallas_call(
        functools.partial(_index_body, qoff_b=qoff_b),
        grid=(NB, nq // _BQ),
        in_specs=[
            pl.BlockSpec((1, _BQ, D_FLOW), lambda b, i: (b, i + qoff_b, 0)),
            pl.BlockSpec((D_FLOW, GCOLS), lambda b, i: (0, 0)),
            pl.BlockSpec((1, GCOLS), lambda b, i: (0, 0)),
            pl.BlockSpec((D_FLOW, GCOLS), lambda b, i: (0, 0)),
            pl.BlockSpec((1, GCOLS), lambda b, i: (0, 0)),
            pl.BlockSpec((D_FLOW, GCOLS), lambda b, i: (0, 0)),
            pl.BlockSpec((1, GCOLS), lambda b, i: (0, 0)),
            pl.BlockSpec((GCOLS, GCOLS), lambda b, i: (0, 0)),
        ] + cspecs,
        out_specs=[
            pl.BlockSpec((1, _BQ, GPQ), lambda b, i: (b, i, 0)),
            pl.BlockSpec((1, _BQ, GPQ), lambda b, i: (b, i, 0)),
        ],
        out_shape=[
            jax.ShapeDtypeStruct((NB, nq, GPQ), jnp.int32),
            jax.ShapeDtypeStruct((NB, nq, GPQ), jnp.uint32),
        ],
    )(flow, wox, box, woy, boy, W_att, b_att.reshape(1, GCOLS), jnp.asarray(_SMAT),
      *consts)


_NTILES = 32
_QC = 9
_CHUNKS = NB * LQ // _QC
_CPW = _CHUNKS // _NTILES
_NIDX = _QC * GPQ
_NSEG = _NIDX // 128


def _sc_body(value_hbm, idx_hbm, w_hbm, out_hbm, idx_v, w_v, rows_v, out_v,
             g0, g1, a0, a1, o0, o1, *, cpw):
    cid = lax.axis_index("c")
    sid = lax.axis_index("s")
    wid = sid * 2 + cid
    gsems = (g0, g1)
    asems = (a0, a1)
    osems = (o0, o1)

    def fetch_aw(slot, i, start):
        ci = wid * cpw + i
        idx_cp = (idx_hbm.at[ci], idx_v.at[pl.ds(slot * _NSEG, _NSEG)],
                  asems[slot])
        w_cp = (w_hbm.at[ci], w_v.at[pl.ds(slot * _NIDX, _NIDX)], asems[slot])
        if start:
            pltpu.async_copy(*idx_cp)
            pltpu.async_copy(*w_cp)
        else:
            pltpu.make_async_copy(*idx_cp).wait()
            pltpu.make_async_copy(*w_cp).wait()

    def gathers(slot, start):
        for j in range(_NSEG):
            cp = (value_hbm.at[idx_v.at[slot * _NSEG + j]],
                  rows_v.at[pl.ds((slot * _NIDX) + j * 128, 128)],
                  gsems[slot])
            if start:
                pltpu.async_copy(*cp)
            else:
                pltpu.make_async_copy(*cp).wait()

    def out_cp(slot, i, start):
        q0 = (wid * cpw + i) * _QC
        cp = (out_v.at[slot], out_hbm.at[pl.ds(q0, _QC)], osems[slot])
        if start:
            pltpu.async_copy(*cp)
        else:
            pltpu.make_async_copy(*cp).wait()

    def compute(slot, i):
        for qi in range(_QC):
            def h_body(h, hc):
                base = slot * _NIDX + qi * GPQ + h * 12
                acc0s, acc1s = [], []
                for c in range(4):
                    wv = w_v[pl.ds(base + c * GCOLS, 16)]
                    a0_ = [jnp.zeros((16,), jnp.float32) for _ in range(2)]
                    a1_ = [jnp.zeros((16,), jnp.float32) for _ in range(2)]
                    for pp in range(3):
                        p = 4 * pp
                        ws = [plsc.bitcast(jnp.take_along_axis(
                            wv, jnp.full((16,), p + j, jnp.int32), axis=0),
                            jnp.bfloat16) for j in range(4)]
                        r0 = base + c * GCOLS + p
                        s = ((rows_v[r0, :] * ws[0]
                              + rows_v[r0 + 1, :] * ws[1])
                             + (rows_v[r0 + 2, :] * ws[2]
                                + rows_v[r0 + 3, :] * ws[3]))
                        va, vb = plsc.unpack(
                            s, format=plsc.PackFormat.INTERLEAVED)
                        a0_[pp % 2] = a0_[pp % 2] + va
                        a1_[pp % 2] = a1_[pp % 2] + vb
                    acc0s.append(a0_[0] + a0_[1])
                    acc1s.append(a1_[0] + a1_[1])
                out_v[slot, qi, pl.ds(h * DH, 16)] = (
                    (acc0s[0] + acc0s[1]) + (acc0s[2] + acc0s[3]))
                out_v[slot, qi, pl.ds(h * DH + 16, 16)] = (
                    (acc1s[0] + acc1s[1]) + (acc1s[2] + acc1s[3]))
                return hc
            lax.fori_loop(0, NH, h_body, 0)

    fetch_aw(0, 0, True)
    fetch_aw(0, 0, False)
    gathers(0, True)
    fetch_aw(1, 1, True)

    def body(k, carry):
        i0 = k * 2
        for b in range(2):
            cur = i0 + b

            @pl.when(cur + 1 < cpw)
            def _():
                fetch_aw(1 - b, cur + 1, False)
                gathers(1 - b, True)

            gathers(b, False)

            @pl.when(cur >= 2)
            def _():
                out_cp(b, cur - 2, False)

            compute(b, cur)
            out_cp(b, cur, True)

            @pl.when(cur + 2 < cpw)
            def _():
                fetch_aw(b, cur + 2, True)
        return carry

    lax.fori_loop(0, cpw // 2, body, 0)
    out_cp(0, cpw - 2, False)
    out_cp(1, cpw - 1, False)


def _sc_sample(value_flat, idx_seg, w_flat, nq):
    cpw = nq // _QC // _NTILES
    mesh = plsc.VectorSubcoreMesh(core_axis_name="c", subcore_axis_name="s",
                                  num_cores=2, num_subcores=16)
    f = functools.partial(
        pl.kernel,
        out_type=jax.ShapeDtypeStruct((nq, D_MODEL), jnp.float32),
        mesh=mesh,
        compiler_params=pltpu.CompilerParams(needs_layout_passes=False,
                                             use_tc_tiling_on_sc=False),
        scratch_types=[
            pltpu.VMEM((2 * _NSEG, 128), jnp.int32),
            pltpu.VMEM((2 * _NIDX + 16,), jnp.uint32),
            pltpu.VMEM((2 * _NIDX, DH), jnp.bfloat16),
            pltpu.VMEM((2, _QC, D_MODEL), jnp.float32),
            pltpu.SemaphoreType.DMA,
            pltpu.SemaphoreType.DMA,
            pltpu.SemaphoreType.DMA,
            pltpu.SemaphoreType.DMA,
            pltpu.SemaphoreType.DMA,
            pltpu.SemaphoreType.DMA,
        ],
    )(functools.partial(_sc_body, cpw=cpw))
    return f(value_flat, idx_seg, w_flat)


_BO = 512


def _tail_body(x_ref, wo_ref, bo_ref, g_ref, be_ref, w1_ref, b1_ref, w2_ref,
               b2_ref, out_ref):
    src2 = jnp.dot(x_ref[...], wo_ref[...], preferred_element_type=jnp.float32) + bo_ref[...]
    mu = jnp.mean(src2, axis=1, keepdims=True)
    var = jnp.mean((src2 - mu) ** 2, axis=1, keepdims=True)
    ln = (src2 - mu) * lax.rsqrt(var + 1e-5) * g_ref[...] + be_ref[...]
    h1 = jnp.maximum(
        jnp.dot(ln, w1_ref[...], preferred_element_type=jnp.float32) + b1_ref[...], 0.0)
    ffn = jnp.dot(h1, w2_ref[...], preferred_element_type=jnp.float32) + b2_ref[...]
    out_ref[...] = src2 + ffn


def _tail(samp, Wo, bo, gamma, beta, W1, b1, W2, b2):
    return pl.pallas_call(
        _tail_body,
        grid=(samp.shape[0] // _BO,),
        in_specs=[
            pl.BlockSpec((_BO, D_MODEL), lambda i: (i, 0)),
            pl.BlockSpec((D_MODEL, D_MODEL), lambda i: (0, 0)),
            pl.BlockSpec((1, D_MODEL), lambda i: (0, 0)),
            pl.BlockSpec((1, D_MODEL), lambda i: (0, 0)),
            pl.BlockSpec((1, D_MODEL), lambda i: (0, 0)),
            pl.BlockSpec((D_MODEL, D_FFN), lambda i: (0, 0)),
            pl.BlockSpec((1, D_FFN), lambda i: (0, 0)),
            pl.BlockSpec((D_FFN, D_MODEL), lambda i: (0, 0)),
            pl.BlockSpec((1, D_MODEL), lambda i: (0, 0)),
        ],
        out_specs=pl.BlockSpec((_BO, D_MODEL), lambda i: (i, 0)),
        out_shape=jax.ShapeDtypeStruct((samp.shape[0], D_MODEL), jnp.float32),
    )(samp, Wo, bo.reshape(1, D_MODEL), gamma.reshape(1, D_MODEL),
      beta.reshape(1, D_MODEL), W1, b1.reshape(1, D_FFN), W2,
      b2.reshape(1, D_MODEL))


def kernel(src, flow, spatial_shapes, level_start_index, Wv, bv, W_off, b_off,
           W_att, b_att, Wo, bo, gamma, beta, W1, b1, W2, b2):
    value = _value_proj(src.reshape(NB * LIN, D_MODEL), Wv[:, _PERM],
                        bv[_PERM])
    value_flat = value.reshape(NB * LIN * NH, DH)
    idx, w = _index_weights(flow, W_off, b_off, W_att, b_att, 0, LQ)
    samp = _sc_sample(value_flat, idx.reshape(_CHUNKS, _NSEG, 128),
                      w.reshape(_CHUNKS, _NIDX), NB * LQ)
    out = _tail(samp, Wo, bo, gamma, beta, W1, b1, W2, b2)
    return out.reshape(NB, LQ, D_MODEL)


# speedup vs baseline: 1.1280x; 1.0342x over previous
import functools

import jax
import jax.numpy as jnp
import numpy as np
from jax import lax
from jax.experimental import pallas as pl
from jax.experimental.pallas import tpu as pltpu
from jax.experimental.pallas import tpu_sc as plsc

D_MODEL = 256
D_FLOW = 128
D_FFN = 1024
NL = 3
NH = 8
NP = 4
_SHAPES = ((96, 96), (48, 48), (24, 24))
_STARTS = (0, 9216, 11520)
NB = 2
LQ = 9216
LIN = 12096
DH = D_MODEL // NH
GCOLS = NH * NL * NP
GPQ = 4 * GCOLS

_l_of_col = np.array([(c % 12) // 4 for c in range(GCOLS)])
_W_COL = np.array([_SHAPES[l][1] for l in _l_of_col], np.float32)[None, :]
_H_COL = np.array([_SHAPES[l][0] for l in _l_of_col], np.float32)[None, :]
_WI_COL = _W_COL.astype(np.int32)
_ST_COL = np.array([_STARTS[l] for l in _l_of_col], np.int32)[None, :]
_HCOL = np.array([c // 12 for c in range(GCOLS)], np.int32)[None, :]
_PERM = np.zeros((D_MODEL,), np.int64)
for _h in range(NH):
    for _c in range(16):
        _PERM[_h * DH + 2 * _c] = _h * DH + _c
        _PERM[_h * DH + 2 * _c + 1] = _h * DH + 16 + _c

_SMAT = np.zeros((GCOLS, GCOLS), np.float32)
for _i in range(GCOLS):
    for _j in range(GCOLS):
        if _i // 12 == _j // 12:
            _SMAT[_i, _j] = 1.0

_BV = 2016


def _value_body(src_ref, wv_ref, bv_ref, out_ref):
    out_ref[...] = (
        jnp.dot(src_ref[...], wv_ref[...], preferred_element_type=jnp.float32)
        + bv_ref[...]
    ).astype(jnp.bfloat16)


def _value_proj(src_flat, Wv, bv):
    return pl.pallas_call(
        _value_body,
        grid=(NB * LIN // _BV,),
        in_specs=[
            pl.BlockSpec((_BV, D_MODEL), lambda i: (i, 0)),
            pl.BlockSpec((D_MODEL, D_MODEL), lambda i: (0, 0)),
            pl.BlockSpec((1, D_MODEL), lambda i: (0, 0)),
        ],
        out_specs=pl.BlockSpec((_BV, D_MODEL), lambda i: (i, 0)),
        out_shape=jax.ShapeDtypeStruct((NB * LIN, D_MODEL), jnp.bfloat16),
    )(src_flat, Wv, bv.reshape(1, D_MODEL))


_BQ = 1024


def _index_body(flow_ref, wox_ref, box_ref, woy_ref, boy_ref, wat_ref,
                bat_ref, smat_ref, wl_ref, hl_ref, wli_ref, st_ref, hc_ref,
                idx_ref, w_ref, *, qoff_b):
    b = pl.program_id(0)
    qb = pl.program_id(1) + qoff_b
    fl = flow_ref[0]
    offx = jnp.dot(fl, wox_ref[...], preferred_element_type=jnp.float32) + box_ref[...]
    offy = jnp.dot(fl, woy_ref[...], preferred_element_type=jnp.float32) + boy_ref[...]
    att = jnp.dot(fl, wat_ref[...], preferred_element_type=jnp.float32) + bat_ref[...]
    m = jnp.max(att, axis=1, keepdims=True)
    e = jnp.exp(att - m)
    s = jnp.dot(e, smat_ref[...], preferred_element_type=jnp.float32)
    aw = e / s

    q = qb * _BQ + lax.broadcasted_iota(jnp.int32, (_BQ, GCOLS), 0)
    xq = (q % 96).astype(jnp.float32)
    yq = (q // 96).astype(jnp.float32)
    wl = wl_ref[...]
    hl = hl_ref[...]
    px = (xq + 0.5) * (wl / 96.0) + offx - 0.5
    py = (yq + 0.5) * (hl / 96.0) + offy - 0.5
    fx = jnp.floor(px)
    tx = px - fx
    fy = jnp.floor(py)
    ty = py - fy
    wlm1 = wl - 1.0
    hlm1 = hl - 1.0
    vx0 = ((fx >= 0.0) & (fx <= wlm1)).astype(jnp.float32)
    vx1 = ((fx + 1.0 >= 0.0) & (fx + 1.0 <= wlm1)).astype(jnp.float32)
    vy0 = ((fy >= 0.0) & (fy <= hlm1)).astype(jnp.float32)
    vy1 = ((fy + 1.0 >= 0.0) & (fy + 1.0 <= hlm1)).astype(jnp.float32)
    ix0 = jnp.clip(fx, 0.0, wlm1).astype(jnp.int32)
    ix1 = jnp.clip(fx + 1.0, 0.0, wlm1).astype(jnp.int32)
    iy0 = jnp.clip(fy, 0.0, hlm1).astype(jnp.int32)
    iy1 = jnp.clip(fy + 1.0, 0.0, hlm1).astype(jnp.int32)
    wx0 = (1.0 - tx) * vx0
    wx1 = tx * vx1
    wy0 = (1.0 - ty) * vy0
    wy1 = ty * vy1

    wli = wli_ref[...]
    base = b * LIN + st_ref[...]
    hc = hc_ref[...]
    corners = ((ix0, iy0, wx0, wy0), (ix1, iy0, wx1, wy0),
               (ix0, iy1, wx0, wy1), (ix1, iy1, wx1, wy1))
    for c, (ix, iy, wx, wy) in enumerate(corners):
        idx_ref[0, :, c * GCOLS:(c + 1) * GCOLS] = (base + iy * wli + ix) * NH + hc
        wb = (aw * wx * wy).astype(jnp.bfloat16)
        wu = lax.bitcast_convert_type(wb, jnp.uint16).astype(jnp.uint32)
        w_ref[0, :, c * GCOLS:(c + 1) * GCOLS] = wu * jnp.uint32(65537)


def _index_weights(flow, W_off, b_off, W_att, b_att, qoff_b, nq):
    wox = W_off[:, 0::2]
    woy = W_off[:, 1::2]
    box = b_off[0::2].reshape(1, GCOLS)
    boy = b_off[1::2].reshape(1, GCOLS)
    consts = [jnp.asarray(_W_COL), jnp.asarray(_H_COL), jnp.asarray(_WI_COL),
              jnp.asarray(_ST_COL), jnp.asarray(_HCOL)]
    cspecs = [pl.BlockSpec((1, GCOLS), lambda b, i: (0, 0)) for _ in consts]
    return pl.pallas_call(
        functools.partial(_index_body, qoff_b=qoff_b),
        grid=(NB, nq // _BQ),
        in_specs=[
            pl.BlockSpec((1, _BQ, D_FLOW), lambda b, i: (b, i + qoff_b, 0)),
            pl.BlockSpec((D_FLOW, GCOLS), lambda b, i: (0, 0)),
            pl.BlockSpec((1, GCOLS), lambda b, i: (0, 0)),
            pl.BlockSpec((D_FLOW, GCOLS), lambda b, i: (0, 0)),
            pl.BlockSpec((1, GCOLS), lambda b, i: (0, 0)),
            pl.BlockSpec((D_FLOW, GCOLS), lambda b, i: (0, 0)),
            pl.BlockSpec((1, GCOLS), lambda b, i: (0, 0)),
            pl.BlockSpec((GCOLS, GCOLS), lambda b, i: (0, 0)),
        ] + cspecs,
        out_specs=[
            pl.BlockSpec((1, _BQ, GPQ), lambda b, i: (b, i, 0)),
            pl.BlockSpec((1, _BQ, GPQ), lambda b, i: (b, i, 0)),
        ],
        out_shape=[
            jax.ShapeDtypeStruct((NB, nq, GPQ), jnp.int32),
            jax.ShapeDtypeStruct((NB, nq, GPQ), jnp.uint32),
        ],
    )(flow, wox, box, woy, boy, W_att, b_att.reshape(1, GCOLS), jnp.asarray(_SMAT),
      *consts)


_NTILES = 32
_QC = 9
_CHUNKS = NB * LQ // _QC
_CPW = _CHUNKS // _NTILES
_NIDX = _QC * GPQ
_NSEG = _NIDX // 128


def _sc_body(value_hbm, idx_hbm, w_hbm, out_hbm, idx_v, w_v, rows_v, out_v,
             g0, g1, a0, a1, o0, o1, *, cpw):
    cid = lax.axis_index("c")
    sid = lax.axis_index("s")
    wid = sid * 2 + cid
    gsems = (g0, g1)
    asems = (a0, a1)
    osems = (o0, o1)

    def fetch_aw(slot, i, start):
        ci = wid * cpw + i
        idx_cp = (idx_hbm.at[ci], idx_v.at[pl.ds(slot * _NSEG, _NSEG)],
                  asems[slot])
        w_cp = (w_hbm.at[ci], w_v.at[pl.ds(slot * _NIDX, _NIDX)], asems[slot])
        if start:
            pltpu.async_copy(*idx_cp)
            pltpu.async_copy(*w_cp)
        else:
            pltpu.make_async_copy(*idx_cp).wait()
            pltpu.make_async_copy(*w_cp).wait()

    def gathers(slot, start):
        for j in range(_NSEG):
            cp = (value_hbm.at[idx_v.at[slot * _NSEG + j]],
                  rows_v.at[pl.ds((slot * _NIDX) + j * 128, 128)],
                  gsems[slot])
            if start:
                pltpu.async_copy(*cp)
            else:
                pltpu.make_async_copy(*cp).wait()

    def out_cp(slot, i, start):
        q0 = (wid * cpw + i) * _QC
        cp = (out_v.at[slot], out_hbm.at[pl.ds(q0, _QC)], osems[slot])
        if start:
            pltpu.async_copy(*cp)
        else:
            pltpu.make_async_copy(*cp).wait()

    def compute(slot, i):
        for qi in range(_QC):
            def h_body(h, hc):
                base = slot * _NIDX + qi * GPQ + h * 12
                acc0s, acc1s = [], []
                for c in range(4):
                    wv = w_v[pl.ds(base + c * GCOLS, 16)]
                    a0_ = [jnp.zeros((16,), jnp.float32) for _ in range(2)]
                    a1_ = [jnp.zeros((16,), jnp.float32) for _ in range(2)]
                    for pp in range(3):
                        p = 4 * pp
                        ws = [plsc.bitcast(jnp.take_along_axis(
                            wv, jnp.full((16,), p + j, jnp.int32), axis=0),
                            jnp.bfloat16) for j in range(4)]
                        r0 = base + c * GCOLS + p
                        s = ((rows_v[r0, :] * ws[0]
                              + rows_v[r0 + 1, :] * ws[1])
                             + (rows_v[r0 + 2, :] * ws[2]
                                + rows_v[r0 + 3, :] * ws[3]))
                        va, vb = plsc.unpack(
                            s, format=plsc.PackFormat.INTERLEAVED)
                        a0_[pp % 2] = a0_[pp % 2] + va
                        a1_[pp % 2] = a1_[pp % 2] + vb
                    acc0s.append(a0_[0] + a0_[1])
                    acc1s.append(a1_[0] + a1_[1])
                out_v[slot, qi, pl.ds(h * DH, 16)] = (
                    (acc0s[0] + acc0s[1]) + (acc0s[2] + acc0s[3]))
                out_v[slot, qi, pl.ds(h * DH + 16, 16)] = (
                    (acc1s[0] + acc1s[1]) + (acc1s[2] + acc1s[3]))
                return hc
            lax.fori_loop(0, NH, h_body, 0)

    fetch_aw(0, 0, True)
    fetch_aw(0, 0, False)
    gathers(0, True)
    fetch_aw(1, 1, True)

    def body(k, carry):
        i0 = k * 2
        for b in range(2):
            cur = i0 + b

            @pl.when(cur + 1 < cpw)
            def _():
                fetch_aw(1 - b, cur + 1, False)
                gathers(1 - b, True)

            gathers(b, False)

            @pl.when(cur >= 2)
            def _():
                out_cp(b, cur - 2, False)

            compute(b, cur)
            out_cp(b, cur, True)

            @pl.when(cur + 2 < cpw)
            def _():
                fetch_aw(b, cur + 2, True)
        return carry

    lax.fori_loop(0, cpw // 2, body, 0)
    out_cp(0, cpw - 2, False)
    out_cp(1, cpw - 1, False)


def _sc_sample(value_flat, idx_seg, w_flat, nq):
    cpw = nq // _QC // _NTILES
    mesh = plsc.VectorSubcoreMesh(core_axis_name="c", subcore_axis_name="s",
                                  num_cores=2, num_subcores=16)
    f = functools.partial(
        pl.kernel,
        out_type=jax.ShapeDtypeStruct((nq, D_MODEL), jnp.float32),
        mesh=mesh,
        compiler_params=pltpu.CompilerParams(needs_layout_passes=False,
                                             use_tc_tiling_on_sc=False),
        scratch_types=[
            pltpu.VMEM((2 * _NSEG, 128), jnp.int32),
            pltpu.VMEM((2 * _NIDX + 16,), jnp.uint32),
            pltpu.VMEM((2 * _NIDX, DH), jnp.bfloat16),
            pltpu.VMEM((2, _QC, D_MODEL), jnp.float32),
            pltpu.SemaphoreType.DMA,
            pltpu.SemaphoreType.DMA,
            pltpu.SemaphoreType.DMA,
            pltpu.SemaphoreType.DMA,
            pltpu.SemaphoreType.DMA,
            pltpu.SemaphoreType.DMA,
        ],
    )(functools.partial(_sc_body, cpw=cpw))
    return f(value_flat, idx_seg, w_flat)


_BO = 1024


def _tail_body(x_ref, wo_ref, bo_ref, g_ref, be_ref, w1_ref, b1_ref, w2_ref,
               b2_ref, out_ref):
    src2 = jnp.dot(x_ref[...], wo_ref[...], preferred_element_type=jnp.float32) + bo_ref[...]
    mu = jnp.mean(src2, axis=1, keepdims=True)
    var = jnp.mean((src2 - mu) ** 2, axis=1, keepdims=True)
    ln = (src2 - mu) * lax.rsqrt(var + 1e-5) * g_ref[...] + be_ref[...]
    h1 = jnp.maximum(
        jnp.dot(ln, w1_ref[...], preferred_element_type=jnp.float32) + b1_ref[...], 0.0)
    ffn = jnp.dot(h1, w2_ref[...], preferred_element_type=jnp.float32) + b2_ref[...]
    out_ref[...] = src2 + ffn


def _tail(samp, Wo, bo, gamma, beta, W1, b1, W2, b2):
    return pl.pallas_call(
        _tail_body,
        grid=(samp.shape[0] // _BO,),
        in_specs=[
            pl.BlockSpec((_BO, D_MODEL), lambda i: (i, 0)),
            pl.BlockSpec((D_MODEL, D_MODEL), lambda i: (0, 0)),
            pl.BlockSpec((1, D_MODEL), lambda i: (0, 0)),
            pl.BlockSpec((1, D_MODEL), lambda i: (0, 0)),
            pl.BlockSpec((1, D_MODEL), lambda i: (0, 0)),
            pl.BlockSpec((D_MODEL, D_FFN), lambda i: (0, 0)),
            pl.BlockSpec((1, D_FFN), lambda i: (0, 0)),
            pl.BlockSpec((D_FFN, D_MODEL), lambda i: (0, 0)),
            pl.BlockSpec((1, D_MODEL), lambda i: (0, 0)),
        ],
        out_specs=pl.BlockSpec((_BO, D_MODEL), lambda i: (i, 0)),
        out_shape=jax.ShapeDtypeStruct((samp.shape[0], D_MODEL), jnp.float32),
    )(samp, Wo, bo.reshape(1, D_MODEL), gamma.reshape(1, D_MODEL),
      beta.reshape(1, D_MODEL), W1, b1.reshape(1, D_FFN), W2,
      b2.reshape(1, D_MODEL))


def kernel(src, flow, spatial_shapes, level_start_index, Wv, bv, W_off, b_off,
           W_att, b_att, Wo, bo, gamma, beta, W1, b1, W2, b2):
    value = _value_proj(src.reshape(NB * LIN, D_MODEL), Wv[:, _PERM],
                        bv[_PERM])
    value_flat = value.reshape(NB * LIN * NH, DH)
    idx, w = _index_weights(flow, W_off, b_off, W_att, b_att, 0, LQ)
    samp = _sc_sample(value_flat, idx.reshape(_CHUNKS, _NSEG, 128),
                      w.reshape(_CHUNKS, _NIDX), NB * LQ)
    out = _tail(samp, Wo, bo, gamma, beta, W1, b1, W2, b2)
    return out.reshape(NB, LQ, D_MODEL)
